# Initial kernel scaffold; baseline (speedup 1.0000x reference)
import jax
import jax.numpy as jnp
from jax.experimental import pallas as pl


def kernel(x, edge_index, batch, W0, b0, g0, be0, W1, b1, g1, be1, W2, b2, g2, be2, We, bemb):
    raise NotImplementedError("write your pallas kernel here")


# speedup vs baseline: 19.3810x; 19.3810x over previous
import functools

import jax
import jax.numpy as jnp
import numpy as np
from jax import lax
from jax.experimental import pallas as pl
from jax.experimental.pallas import tpu as pltpu
from jax.experimental.pallas import tpu_sc as plsc

N = 10000
NP = 10240
D = 128
H = 128
EMB = 64
E = 320000
B = 16

NC = 2
NS = 16
NW = NC * NS
LANE = 128
EC = 79
EPW = EC * LANE
EP = NW * EPW
ROWS_PER_SUB = NP // NS

BN_C = float(1.0 / np.sqrt(1.0 + 1e-5))

_MESH = plsc.VectorSubcoreMesh(core_axis_name="c", subcore_axis_name="s")


@functools.partial(
    pl.kernel,
    out_type=jax.ShapeDtypeStruct((NC, NP), jnp.float32),
    mesh=_MESH,
    scratch_types=[
        pltpu.VMEM((EC, LANE), jnp.int32),
        pltpu.VMEM((LANE,), jnp.float32),
        pltpu.VMEM_SHARED((NP,), jnp.float32),
    ],
)
def _deg_kernel(dst_hbm, zeros1_hbm, out_hbm, dst_v, ones_v, acc):
    c = lax.axis_index("c")
    s = lax.axis_index("s")
    wid = c * NS + s
    for l in range(LANE // 16):
        ones_v[pl.ds(l * 16, 16)] = jnp.ones((16,), jnp.float32)
    pltpu.sync_copy(zeros1_hbm.at[pl.ds(s * ROWS_PER_SUB, ROWS_PER_SUB)],
                    acc.at[pl.ds(s * ROWS_PER_SUB, ROWS_PER_SUB)])
    pltpu.sync_copy(dst_hbm.at[wid], dst_v)
    plsc.subcore_barrier()

    @pl.loop(0, EC)
    def _(j):
        pltpu.sync_copy(ones_v, acc.at[dst_v.at[j]], add=True)

    plsc.subcore_barrier()
    pltpu.sync_copy(acc.at[pl.ds(s * ROWS_PER_SUB, ROWS_PER_SUB)],
                    out_hbm.at[c, pl.ds(s * ROWS_PER_SUB, ROWS_PER_SUB)])


@functools.partial(
    pl.kernel,
    out_type=jax.ShapeDtypeStruct((NC, NP, H), jnp.float32),
    mesh=_MESH,
    scratch_types=[
        pltpu.VMEM((EC, LANE), jnp.int32),
        pltpu.VMEM((EC, LANE), jnp.int32),
        pltpu.VMEM((LANE, H), jnp.float32),
        pltpu.VMEM_SHARED((NP, H), jnp.float32),
        pltpu.SemaphoreType.DMA,
    ],
)
def _agg_kernel(hs_hbm, src_hbm, dst_hbm, zeros2_hbm, out_hbm,
                src_v, dst_v, rows_v, acc, sem):
    c = lax.axis_index("c")
    s = lax.axis_index("s")
    wid = c * NS + s
    pltpu.sync_copy(zeros2_hbm.at[pl.ds(s * ROWS_PER_SUB, ROWS_PER_SUB)],
                    acc.at[pl.ds(s * ROWS_PER_SUB, ROWS_PER_SUB)])
    pltpu.sync_copy(src_hbm.at[wid], src_v)
    pltpu.sync_copy(dst_hbm.at[wid], dst_v)
    plsc.subcore_barrier()

    @pl.loop(0, EC)
    def _(j):
        pltpu.async_copy(hs_hbm.at[src_v.at[j]], rows_v, sem).wait()
        pltpu.sync_copy(rows_v, acc.at[dst_v.at[j]], add=True)

    plsc.subcore_barrier()
    pltpu.sync_copy(acc.at[pl.ds(s * ROWS_PER_SUB, ROWS_PER_SUB)],
                    out_hbm.at[c, pl.ds(s * ROWS_PER_SUB, ROWS_PER_SUB)])


def _tc_first(x_ref, w_ref, degp_ref, h_ref, hs_ref, dinv_ref):
    deg = degp_ref[0] + degp_ref[1] + 1.0
    dinv = lax.rsqrt(deg)
    h = jnp.dot(x_ref[...], w_ref[...], preferred_element_type=jnp.float32)
    h_ref[...] = h
    hs_ref[...] = h * dinv
    dinv_ref[...] = dinv


def _tc_mid(h_ref, aggp_ref, dinv_ref, b_ref, g_ref, be_ref, w_ref,
            hn_ref, hsn_ref):
    dinv = dinv_ref[...]
    agg = aggp_ref[0] + aggp_ref[1]
    h = h_ref[...]
    out = dinv * agg + (dinv * dinv) * h + b_ref[...]
    a = jnp.maximum(out * (BN_C * g_ref[...]) + be_ref[...], 0.0)
    hn = jnp.dot(a, w_ref[...], preferred_element_type=jnp.float32)
    hn_ref[...] = hn
    hsn_ref[...] = hn * dinv


def _tc_final(h_ref, aggp_ref, dinv_ref, b_ref, g_ref, be_ref, batch_ref,
              we_ref, bemb_ref, out_ref):
    dinv = dinv_ref[...]
    agg = aggp_ref[0] + aggp_ref[1]
    h = h_ref[...]
    out = dinv * agg + (dinv * dinv) * h + b_ref[...]
    a = jnp.maximum(out * (BN_C * g_ref[...]) + be_ref[...], 0.0)
    iot = lax.broadcasted_iota(jnp.int32, (B, NP), 0)
    onehot = (batch_ref[...] == iot).astype(jnp.float32)
    seg = jnp.dot(onehot, a, preferred_element_type=jnp.float32)
    cnt = jnp.sum(onehot, axis=1, keepdims=True)
    pooled = seg / jnp.maximum(cnt, 1.0)
    out_ref[...] = (jnp.dot(pooled, we_ref[...],
                            preferred_element_type=jnp.float32) + bemb_ref[...])


_first_call = pl.pallas_call(
    _tc_first,
    out_shape=[
        jax.ShapeDtypeStruct((NP, H), jnp.float32),
        jax.ShapeDtypeStruct((NP, H), jnp.float32),
        jax.ShapeDtypeStruct((NP, 1), jnp.float32),
    ],
)

_mid_call = pl.pallas_call(
    _tc_mid,
    out_shape=[
        jax.ShapeDtypeStruct((NP, H), jnp.float32),
        jax.ShapeDtypeStruct((NP, H), jnp.float32),
    ],
)

_final_call = pl.pallas_call(
    _tc_final,
    out_shape=jax.ShapeDtypeStruct((B, EMB), jnp.float32),
)


def kernel(x, edge_index, batch, W0, b0, g0, be0, W1, b1, g1, be1,
           W2, b2, g2, be2, We, bemb):
    f32 = jnp.float32
    pad_n = NP - N
    x_p = jnp.concatenate([x, jnp.zeros((pad_n, D), f32)], axis=0)
    batch_p = jnp.concatenate(
        [batch, jnp.full((pad_n,), B, jnp.int32)]).reshape(1, NP)

    pad_e = EP - E
    pad_idx = (N + (jnp.arange(pad_e, dtype=jnp.int32) % pad_n))
    src_a = jnp.concatenate([edge_index[0], pad_idx]).reshape(NW, EC, LANE)
    dst_a = jnp.concatenate([edge_index[1], pad_idx]).reshape(NW, EC, LANE)

    z1 = jnp.zeros((NP,), f32)
    z2 = jnp.zeros((NP, H), f32)

    deg_parts = _deg_kernel(dst_a, z1)
    degp = deg_parts.reshape(NC, NP, 1)

    h0, hs0, dinv = _first_call(x_p, W0, degp)
    agg0 = _agg_kernel(hs0, src_a, dst_a, z2)
    h1, hs1 = _mid_call(h0, agg0, dinv, b0.reshape(1, H), g0.reshape(1, H),
                        be0.reshape(1, H), W1)
    agg1 = _agg_kernel(hs1, src_a, dst_a, z2)
    h2, hs2 = _mid_call(h1, agg1, dinv, b1.reshape(1, H), g1.reshape(1, H),
                        be1.reshape(1, H), W2)
    agg2 = _agg_kernel(hs2, src_a, dst_a, z2)
    out = _final_call(h2, agg2, dinv, b2.reshape(1, H), g2.reshape(1, H),
                      be2.reshape(1, H), batch_p, We, bemb.reshape(1, EMB))
    return out


# speedup vs baseline: 25.7882x; 1.3306x over previous
import functools

import jax
import jax.numpy as jnp
import numpy as np
from jax import lax
from jax.experimental import pallas as pl
from jax.experimental.pallas import tpu as pltpu
from jax.experimental.pallas import tpu_sc as plsc

N = 10000
NP = 10240
D = 128
H = 128
EMB = 64
E = 320000
B = 16

NC = 2
NS = 16
NW = NC * NS
LANE = 64
EC = 160
EPW = EC * LANE
EP = NW * EPW
ROWS_PER_SUB = NP // NS

BN_C = float(1.0 / np.sqrt(1.0 + 1e-5))

_MESH = plsc.VectorSubcoreMesh(core_axis_name="c", subcore_axis_name="s")


@functools.partial(
    pl.kernel,
    out_type=jax.ShapeDtypeStruct((NC, NP), jnp.float32),
    mesh=_MESH,
    scratch_types=[
        pltpu.VMEM((EC, LANE), jnp.int32),
        pltpu.VMEM((LANE,), jnp.float32),
        pltpu.VMEM_SHARED((NP,), jnp.float32),
    ],
)
def _deg_kernel(dst_hbm, zeros1_hbm, out_hbm, dst_v, ones_v, acc):
    c = lax.axis_index("c")
    s = lax.axis_index("s")
    wid = c * NS + s
    for l in range(LANE // 16):
        ones_v[pl.ds(l * 16, 16)] = jnp.ones((16,), jnp.float32)
    pltpu.sync_copy(zeros1_hbm.at[pl.ds(s * ROWS_PER_SUB, ROWS_PER_SUB)],
                    acc.at[pl.ds(s * ROWS_PER_SUB, ROWS_PER_SUB)])
    pltpu.sync_copy(dst_hbm.at[wid], dst_v)
    plsc.subcore_barrier()

    @pl.loop(0, EC)
    def _(j):
        pltpu.sync_copy(ones_v, acc.at[dst_v.at[j]], add=True)

    plsc.subcore_barrier()
    pltpu.sync_copy(acc.at[pl.ds(s * ROWS_PER_SUB, ROWS_PER_SUB)],
                    out_hbm.at[c, pl.ds(s * ROWS_PER_SUB, ROWS_PER_SUB)])


@functools.partial(
    pl.kernel,
    out_type=jax.ShapeDtypeStruct((NC, NP, H), jnp.float32),
    mesh=_MESH,
    scratch_types=[
        pltpu.VMEM((EC // 4, LANE), jnp.int32),
        pltpu.VMEM((EC // 4, LANE), jnp.int32),
        pltpu.VMEM((LANE, H), jnp.float32),
        pltpu.VMEM((LANE, H), jnp.float32),
        pltpu.VMEM((LANE, H), jnp.float32),
        pltpu.VMEM((LANE, H), jnp.float32),
        pltpu.VMEM_SHARED((NP, H), jnp.float32),
        pltpu.SemaphoreType.DMA,
        pltpu.SemaphoreType.DMA,
        pltpu.SemaphoreType.DMA,
        pltpu.SemaphoreType.DMA,
        pltpu.SemaphoreType.DMA,
        pltpu.SemaphoreType.DMA,
        pltpu.SemaphoreType.DMA,
        pltpu.SemaphoreType.DMA,
        pltpu.SemaphoreType.DMA,
    ],
)
def _agg_kernel(hs_hbm, src_hbm, dst_hbm, zeros2_hbm, out_hbm,
                src_v, dst_v, rows0, rows1, rows2, rows3, acc, semz,
                sg0, sg1, sg2, sg3, ss0, ss1, ss2, ss3):
    c = lax.axis_index("c")
    s = lax.axis_index("s")
    wid = c * NS + s
    rows = (rows0, rows1, rows2, rows3)
    sg = (sg0, sg1, sg2, sg3)
    ss = (ss0, ss1, ss2, ss3)
    PC = EC // 4
    zcp = pltpu.async_copy(
        zeros2_hbm.at[pl.ds(s * ROWS_PER_SUB, ROWS_PER_SUB)],
        acc.at[pl.ds(s * ROWS_PER_SUB, ROWS_PER_SUB)], semz)
    for p in range(4):
        pltpu.sync_copy(src_hbm.at[wid, pl.ds(p * PC, PC)], src_v)
        pltpu.sync_copy(dst_hbm.at[wid, pl.ds(p * PC, PC)], dst_v)
        if p == 0:
            zcp.wait()
            plsc.subcore_barrier()

        @pl.loop(0, PC, step=4)
        def _(j):
            for b in range(4):
                @pl.when(j > 0)
                def _():
                    pltpu.make_async_copy(
                        rows[b], acc.at[pl.ds(0, LANE)], ss[b]).wait()
                pltpu.async_copy(hs_hbm.at[src_v.at[j + b]], rows[b], sg[b])
            for b in range(4):
                pltpu.make_async_copy(
                    hs_hbm.at[src_v.at[j + b]], rows[b], sg[b]).wait()
                pltpu.async_copy(rows[b], acc.at[dst_v.at[j + b]], ss[b],
                                 add=True)

        for b in range(4):
            pltpu.make_async_copy(rows[b], acc.at[pl.ds(0, LANE)], ss[b]).wait()
    plsc.subcore_barrier()
    pltpu.sync_copy(acc.at[pl.ds(s * ROWS_PER_SUB, ROWS_PER_SUB)],
                    out_hbm.at[c, pl.ds(s * ROWS_PER_SUB, ROWS_PER_SUB)])


def _tc_first(x_ref, w_ref, degp_ref, hs_ref, dinv_ref):
    deg = degp_ref[0] + degp_ref[1] + 1.0
    dinv = lax.rsqrt(deg)
    h = jnp.dot(x_ref[...], w_ref[...], preferred_element_type=jnp.float32)
    hs = h * dinv
    hs_ref[0] = hs
    hs_ref[1] = hs
    dinv_ref[...] = dinv


def _tc_mid(hs_ref, aggp_ref, dinv_ref, b_ref, g_ref, be_ref, w_ref, hsn_ref):
    dinv = dinv_ref[...]
    out = dinv * (aggp_ref[0] + aggp_ref[1] + hs_ref[0]) + b_ref[...]
    a = jnp.maximum(out * (BN_C * g_ref[...]) + be_ref[...], 0.0)
    hsn = jnp.dot(a, w_ref[...], preferred_element_type=jnp.float32) * dinv
    hsn_ref[0] = hsn
    hsn_ref[1] = hsn


def _tc_final(hs_ref, aggp_ref, dinv_ref, b_ref, g_ref, be_ref, batch_ref,
              we_ref, bemb_ref, out_ref):
    dinv = dinv_ref[...]
    out = dinv * (aggp_ref[0] + aggp_ref[1] + hs_ref[0]) + b_ref[...]
    a = jnp.maximum(out * (BN_C * g_ref[...]) + be_ref[...], 0.0)
    iot = lax.broadcasted_iota(jnp.int32, (B, NP), 0)
    onehot = (batch_ref[...] == iot).astype(jnp.float32)
    seg = jnp.dot(onehot, a, preferred_element_type=jnp.float32)
    cnt = jnp.sum(onehot, axis=1, keepdims=True)
    pooled = seg / jnp.maximum(cnt, 1.0)
    out_ref[...] = (jnp.dot(pooled, we_ref[...],
                            preferred_element_type=jnp.float32) + bemb_ref[...])


_first_call = pl.pallas_call(
    _tc_first,
    out_shape=[
        jax.ShapeDtypeStruct((NC, NP, H), jnp.float32),
        jax.ShapeDtypeStruct((NP, 1), jnp.float32),
    ],
)

_mid_call = pl.pallas_call(
    _tc_mid,
    out_shape=jax.ShapeDtypeStruct((NC, NP, H), jnp.float32),
)

_final_call = pl.pallas_call(
    _tc_final,
    out_shape=jax.ShapeDtypeStruct((B, EMB), jnp.float32),
)


def kernel(x, edge_index, batch, W0, b0, g0, be0, W1, b1, g1, be1,
           W2, b2, g2, be2, We, bemb):
    f32 = jnp.float32
    pad_n = NP - N
    x_p = jnp.concatenate([x, jnp.zeros((pad_n, D), f32)], axis=0)
    batch_p = jnp.concatenate(
        [batch, jnp.full((pad_n,), B, jnp.int32)]).reshape(1, NP)

    pad_e = EP - E
    pad_idx = (N + (jnp.arange(pad_e, dtype=jnp.int32) % pad_n))
    src_a = jnp.concatenate([edge_index[0], pad_idx]).reshape(NW, EC, LANE)
    src_a = src_a + (jnp.arange(NW, dtype=jnp.int32) // NS * NP)[:, None, None]
    dst_a = jnp.concatenate([edge_index[1], pad_idx]).reshape(NW, EC, LANE)

    z1 = jnp.zeros((NP,), f32)
    z2 = jnp.zeros((NP, H), f32)

    deg_parts = _deg_kernel(dst_a, z1)
    degp = deg_parts.reshape(NC, NP, 1)

    hs0, dinv = _first_call(x_p, W0, degp)
    agg0 = _agg_kernel(hs0.reshape(NC * NP, H), src_a, dst_a, z2)
    hs1 = _mid_call(hs0, agg0, dinv, b0.reshape(1, H), g0.reshape(1, H),
                    be0.reshape(1, H), W1)
    agg1 = _agg_kernel(hs1.reshape(NC * NP, H), src_a, dst_a, z2)
    hs2 = _mid_call(hs1, agg1, dinv, b1.reshape(1, H), g1.reshape(1, H),
                    be1.reshape(1, H), W2)
    agg2 = _agg_kernel(hs2.reshape(NC * NP, H), src_a, dst_a, z2)
    out = _final_call(hs2, agg2, dinv, b2.reshape(1, H), g2.reshape(1, H),
                      be2.reshape(1, H), batch_p, We, bemb.reshape(1, EMB))
    return out


# speedup vs baseline: 26.2251x; 1.0169x over previous
import functools

import jax
import jax.numpy as jnp
import numpy as np
from jax import lax
from jax.experimental import pallas as pl
from jax.experimental.pallas import tpu as pltpu
from jax.experimental.pallas import tpu_sc as plsc

N = 10000
NP = 10240
D = 128
H = 128
EMB = 64
E = 320000
B = 16

NC = 2
NS = 16
NW = NC * NS
LANE = 64
EC = 160
EPW = EC * LANE
EP = NW * EPW
ROWS_PER_SUB = NP // NS

BN_C = float(1.0 / np.sqrt(1.0 + 1e-5))

_MESH = plsc.VectorSubcoreMesh(core_axis_name="c", subcore_axis_name="s")


@functools.partial(
    pl.kernel,
    out_type=jax.ShapeDtypeStruct((NC, NP), jnp.float32),
    mesh=_MESH,
    scratch_types=[
        pltpu.VMEM((EC, LANE), jnp.int32),
        pltpu.VMEM((LANE,), jnp.float32),
        pltpu.VMEM_SHARED((NP,), jnp.float32),
    ],
)
def _deg_kernel(dst_hbm, zeros1_hbm, out_hbm, dst_v, ones_v, acc):
    c = lax.axis_index("c")
    s = lax.axis_index("s")
    wid = c * NS + s
    for l in range(LANE // 16):
        ones_v[pl.ds(l * 16, 16)] = jnp.ones((16,), jnp.float32)
    pltpu.sync_copy(zeros1_hbm.at[pl.ds(s * ROWS_PER_SUB, ROWS_PER_SUB)],
                    acc.at[pl.ds(s * ROWS_PER_SUB, ROWS_PER_SUB)])
    pltpu.sync_copy(dst_hbm.at[wid], dst_v)
    plsc.subcore_barrier()

    @pl.loop(0, EC)
    def _(j):
        pltpu.sync_copy(ones_v, acc.at[dst_v.at[j]], add=True)

    plsc.subcore_barrier()
    pltpu.sync_copy(acc.at[pl.ds(s * ROWS_PER_SUB, ROWS_PER_SUB)],
                    out_hbm.at[c, pl.ds(s * ROWS_PER_SUB, ROWS_PER_SUB)])


@functools.partial(
    pl.kernel,
    out_type=jax.ShapeDtypeStruct((NC, NP, H), jnp.float32),
    mesh=_MESH,
    scratch_types=[
        pltpu.VMEM((EC // 4, LANE), jnp.int32),
        pltpu.VMEM((EC // 4, LANE), jnp.int32),
        pltpu.VMEM((LANE, H), jnp.float32),
        pltpu.VMEM((LANE, H), jnp.float32),
        pltpu.VMEM((LANE, H), jnp.float32),
        pltpu.VMEM((LANE, H), jnp.float32),
        pltpu.VMEM_SHARED((NP, H), jnp.float32),
        pltpu.SemaphoreType.DMA,
        pltpu.SemaphoreType.DMA,
        pltpu.SemaphoreType.DMA,
        pltpu.SemaphoreType.DMA,
        pltpu.SemaphoreType.DMA,
        pltpu.SemaphoreType.DMA,
        pltpu.SemaphoreType.DMA,
        pltpu.SemaphoreType.DMA,
        pltpu.SemaphoreType.DMA,
    ],
)
def _agg_kernel(hs_hbm, src_hbm, dst_hbm, zeros2_hbm, out_hbm,
                src_v, dst_v, rows0, rows1, rows2, rows3, acc, semz,
                sg0, sg1, sg2, sg3, ss0, ss1, ss2, ss3):
    c = lax.axis_index("c")
    s = lax.axis_index("s")
    wid = c * NS + s
    rows = (rows0, rows1, rows2, rows3)
    sg = (sg0, sg1, sg2, sg3)
    ss = (ss0, ss1, ss2, ss3)
    PC = EC // 4
    zcp = pltpu.async_copy(
        zeros2_hbm.at[pl.ds(s * ROWS_PER_SUB, ROWS_PER_SUB)],
        acc.at[pl.ds(s * ROWS_PER_SUB, ROWS_PER_SUB)], semz)
    for p in range(4):
        pltpu.sync_copy(src_hbm.at[wid, pl.ds(p * PC, PC)], src_v)
        pltpu.sync_copy(dst_hbm.at[wid, pl.ds(p * PC, PC)], dst_v)
        if p == 0:
            zcp.wait()
            plsc.subcore_barrier()

        @pl.loop(0, PC, step=4)
        def _(j):
            for b in range(4):
                @pl.when(j > 0)
                def _():
                    pltpu.make_async_copy(
                        rows[b], acc.at[pl.ds(0, LANE)], ss[b]).wait()
                pltpu.async_copy(hs_hbm.at[src_v.at[j + b]], rows[b], sg[b])
            for b in range(4):
                pltpu.make_async_copy(
                    hs_hbm.at[src_v.at[j + b]], rows[b], sg[b]).wait()
                pltpu.async_copy(rows[b], acc.at[dst_v.at[j + b]], ss[b],
                                 add=True)

        for b in range(4):
            pltpu.make_async_copy(rows[b], acc.at[pl.ds(0, LANE)], ss[b]).wait()
    plsc.subcore_barrier()
    pltpu.sync_copy(acc.at[pl.ds(s * ROWS_PER_SUB, ROWS_PER_SUB)],
                    out_hbm.at[c, pl.ds(s * ROWS_PER_SUB, ROWS_PER_SUB)])


def _tc_first(x_ref, w_ref, degp_ref, hs_ref, dinv_ref):
    deg = degp_ref[0] + degp_ref[1] + 1.0
    dinv = lax.rsqrt(deg)
    h = jnp.dot(x_ref[...], w_ref[...], preferred_element_type=jnp.float32)
    hs_ref[...] = h * dinv
    dinv_ref[...] = dinv


def _tc_mid(hs_ref, aggp_ref, dinv_ref, b_ref, g_ref, be_ref, w_ref, hsn_ref):
    dinv = dinv_ref[...]
    out = dinv * (aggp_ref[0] + aggp_ref[1] + hs_ref[...]) + b_ref[...]
    a = jnp.maximum(out * (BN_C * g_ref[...]) + be_ref[...], 0.0)
    hsn_ref[...] = jnp.dot(a, w_ref[...],
                           preferred_element_type=jnp.float32) * dinv


def _tc_final(hs_ref, aggp_ref, dinv_ref, b_ref, g_ref, be_ref, batch_ref,
              we_ref, bemb_ref, out_ref):
    dinv = dinv_ref[...]
    out = dinv * (aggp_ref[0] + aggp_ref[1] + hs_ref[...]) + b_ref[...]
    a = jnp.maximum(out * (BN_C * g_ref[...]) + be_ref[...], 0.0)
    iot = lax.broadcasted_iota(jnp.int32, (B, NP), 0)
    onehot = (batch_ref[...] == iot).astype(jnp.float32)
    seg = jnp.dot(onehot, a, preferred_element_type=jnp.float32)
    cnt = jnp.sum(onehot, axis=1, keepdims=True)
    pooled = seg / jnp.maximum(cnt, 1.0)
    out_ref[...] = (jnp.dot(pooled, we_ref[...],
                            preferred_element_type=jnp.float32) + bemb_ref[...])


_first_call = pl.pallas_call(
    _tc_first,
    out_shape=[
        jax.ShapeDtypeStruct((NP, H), jnp.float32),
        jax.ShapeDtypeStruct((NP, 1), jnp.float32),
    ],
)

_mid_call = pl.pallas_call(
    _tc_mid,
    out_shape=jax.ShapeDtypeStruct((NP, H), jnp.float32),
)

_final_call = pl.pallas_call(
    _tc_final,
    out_shape=jax.ShapeDtypeStruct((B, EMB), jnp.float32),
)


def kernel(x, edge_index, batch, W0, b0, g0, be0, W1, b1, g1, be1,
           W2, b2, g2, be2, We, bemb):
    f32 = jnp.float32
    pad_n = NP - N
    x_p = jnp.concatenate([x, jnp.zeros((pad_n, D), f32)], axis=0)
    batch_p = jnp.concatenate(
        [batch, jnp.full((pad_n,), B, jnp.int32)]).reshape(1, NP)

    pad_e = EP - E
    pad_idx = (N + (jnp.arange(pad_e, dtype=jnp.int32) % pad_n))
    src_a = jnp.concatenate([edge_index[0], pad_idx]).reshape(NW, EC, LANE)
    dst_a = jnp.concatenate([edge_index[1], pad_idx]).reshape(NW, EC, LANE)

    z1 = jnp.zeros((NP,), f32)
    z2 = jnp.zeros((NP, H), f32)

    deg_parts = _deg_kernel(dst_a, z1)
    degp = deg_parts.reshape(NC, NP, 1)

    hs0, dinv = _first_call(x_p, W0, degp)
    agg0 = _agg_kernel(hs0, src_a, dst_a, z2)
    hs1 = _mid_call(hs0, agg0, dinv, b0.reshape(1, H), g0.reshape(1, H),
                    be0.reshape(1, H), W1)
    agg1 = _agg_kernel(hs1, src_a, dst_a, z2)
    hs2 = _mid_call(hs1, agg1, dinv, b1.reshape(1, H), g1.reshape(1, H),
                    be1.reshape(1, H), W2)
    agg2 = _agg_kernel(hs2, src_a, dst_a, z2)
    out = _final_call(hs2, agg2, dinv, b2.reshape(1, H), g2.reshape(1, H),
                      be2.reshape(1, H), batch_p, We, bemb.reshape(1, EMB))
    return out


# speedup vs baseline: 26.8081x; 1.0222x over previous
import functools

import jax
import jax.numpy as jnp
import numpy as np
from jax import lax
from jax.experimental import pallas as pl
from jax.experimental.pallas import tpu as pltpu
from jax.experimental.pallas import tpu_sc as plsc

N = 10000
NP = 10240
D = 128
H = 128
EMB = 64
E = 320000
B = 16

NC = 2
NS = 16
NW = NC * NS
LANE = 64
EC = 160
EPW = EC * LANE
EP = NW * EPW
ROWS_PER_SUB = NP // NS

BN_C = float(1.0 / np.sqrt(1.0 + 1e-5))

_MESH = plsc.VectorSubcoreMesh(core_axis_name="c", subcore_axis_name="s")


@functools.partial(
    pl.kernel,
    out_type=jax.ShapeDtypeStruct((NC, NP), jnp.float32),
    mesh=_MESH,
    scratch_types=[
        pltpu.VMEM((EC, LANE), jnp.int32),
        pltpu.VMEM((LANE,), jnp.float32),
        pltpu.VMEM_SHARED((NP,), jnp.float32),
        pltpu.SemaphoreType.DMA,
        pltpu.SemaphoreType.DMA,
        pltpu.SemaphoreType.DMA,
        pltpu.SemaphoreType.DMA,
    ],
)
def _deg_kernel(dst_hbm, zeros1_hbm, out_hbm, dst_v, ones_v, acc,
                sd0, sd1, sd2, sd3):
    c = lax.axis_index("c")
    s = lax.axis_index("s")
    wid = c * NS + s
    sd = (sd0, sd1, sd2, sd3)
    for l in range(LANE // 16):
        ones_v[pl.ds(l * 16, 16)] = jnp.ones((16,), jnp.float32)
    pltpu.sync_copy(zeros1_hbm.at[pl.ds(s * ROWS_PER_SUB, ROWS_PER_SUB)],
                    acc.at[pl.ds(s * ROWS_PER_SUB, ROWS_PER_SUB)])
    pltpu.sync_copy(dst_hbm.at[wid], dst_v)
    plsc.subcore_barrier()

    @pl.loop(0, EC, step=4)
    def _(j):
        for b in range(4):
            @pl.when(j > 0)
            def _():
                pltpu.make_async_copy(
                    ones_v, acc.at[pl.ds(0, LANE)], sd[b]).wait()
            pltpu.async_copy(ones_v, acc.at[dst_v.at[j + b]], sd[b], add=True)

    for b in range(4):
        pltpu.make_async_copy(ones_v, acc.at[pl.ds(0, LANE)], sd[b]).wait()
    plsc.subcore_barrier()
    pltpu.sync_copy(acc.at[pl.ds(s * ROWS_PER_SUB, ROWS_PER_SUB)],
                    out_hbm.at[c, pl.ds(s * ROWS_PER_SUB, ROWS_PER_SUB)])


@functools.partial(
    pl.kernel,
    out_type=jax.ShapeDtypeStruct((NC, NP, H), jnp.float32),
    mesh=_MESH,
    scratch_types=[
        pltpu.VMEM((EC // 4, LANE), jnp.int32),
        pltpu.VMEM((EC // 4, LANE), jnp.int32),
        pltpu.VMEM((LANE, H), jnp.float32),
        pltpu.VMEM((LANE, H), jnp.float32),
        pltpu.VMEM((LANE, H), jnp.float32),
        pltpu.VMEM((LANE, H), jnp.float32),
        pltpu.VMEM_SHARED((NP, H), jnp.float32),
        pltpu.SemaphoreType.DMA,
        pltpu.SemaphoreType.DMA,
        pltpu.SemaphoreType.DMA,
        pltpu.SemaphoreType.DMA,
        pltpu.SemaphoreType.DMA,
        pltpu.SemaphoreType.DMA,
        pltpu.SemaphoreType.DMA,
        pltpu.SemaphoreType.DMA,
        pltpu.SemaphoreType.DMA,
    ],
)
def _agg_kernel(hs_hbm, src_hbm, dst_hbm, zeros2_hbm, out_hbm,
                src_v, dst_v, rows0, rows1, rows2, rows3, acc, semz,
                sg0, sg1, sg2, sg3, ss0, ss1, ss2, ss3):
    c = lax.axis_index("c")
    s = lax.axis_index("s")
    wid = c * NS + s
    rows = (rows0, rows1, rows2, rows3)
    sg = (sg0, sg1, sg2, sg3)
    ss = (ss0, ss1, ss2, ss3)
    PC = EC // 4
    zcp = pltpu.async_copy(
        zeros2_hbm.at[pl.ds(s * ROWS_PER_SUB, ROWS_PER_SUB)],
        acc.at[pl.ds(s * ROWS_PER_SUB, ROWS_PER_SUB)], semz)
    for p in range(4):
        pltpu.sync_copy(src_hbm.at[wid, pl.ds(p * PC, PC)], src_v)
        pltpu.sync_copy(dst_hbm.at[wid, pl.ds(p * PC, PC)], dst_v)
        if p == 0:
            zcp.wait()
            plsc.subcore_barrier()

        @pl.loop(0, PC, step=4)
        def _(j):
            for b in range(4):
                @pl.when(j > 0)
                def _():
                    pltpu.make_async_copy(
                        rows[b], acc.at[pl.ds(0, LANE)], ss[b]).wait()
                pltpu.async_copy(hs_hbm.at[src_v.at[j + b]], rows[b], sg[b])
            for b in range(4):
                pltpu.make_async_copy(
                    hs_hbm.at[src_v.at[j + b]], rows[b], sg[b]).wait()
                pltpu.async_copy(rows[b], acc.at[dst_v.at[j + b]], ss[b],
                                 add=True)

        for b in range(4):
            pltpu.make_async_copy(rows[b], acc.at[pl.ds(0, LANE)], ss[b]).wait()
    plsc.subcore_barrier()
    pltpu.sync_copy(acc.at[pl.ds(s * ROWS_PER_SUB, ROWS_PER_SUB)],
                    out_hbm.at[c, pl.ds(s * ROWS_PER_SUB, ROWS_PER_SUB)])


def _tc_first(x_ref, w_ref, degp_ref, hs_ref, dinv_ref):
    deg = degp_ref[0] + degp_ref[1] + 1.0
    dinv = lax.rsqrt(deg)
    h = jnp.dot(x_ref[...], w_ref[...], preferred_element_type=jnp.float32)
    hs_ref[...] = h * dinv
    dinv_ref[...] = dinv


def _tc_mid(hs_ref, aggp_ref, dinv_ref, b_ref, g_ref, be_ref, w_ref, hsn_ref):
    dinv = dinv_ref[...]
    out = dinv * (aggp_ref[0] + aggp_ref[1] + hs_ref[...]) + b_ref[...]
    a = jnp.maximum(out * (BN_C * g_ref[...]) + be_ref[...], 0.0)
    hsn_ref[...] = jnp.dot(a, w_ref[...],
                           preferred_element_type=jnp.float32) * dinv


def _tc_final(hs_ref, aggp_ref, dinv_ref, b_ref, g_ref, be_ref, batch_ref,
              we_ref, bemb_ref, out_ref):
    dinv = dinv_ref[...]
    out = dinv * (aggp_ref[0] + aggp_ref[1] + hs_ref[...]) + b_ref[...]
    a = jnp.maximum(out * (BN_C * g_ref[...]) + be_ref[...], 0.0)
    iot = lax.broadcasted_iota(jnp.int32, (B, NP), 0)
    onehot = (batch_ref[...] == iot).astype(jnp.float32)
    seg = jnp.dot(onehot, a, preferred_element_type=jnp.float32)
    cnt = jnp.sum(onehot, axis=1, keepdims=True)
    pooled = seg / jnp.maximum(cnt, 1.0)
    out_ref[...] = (jnp.dot(pooled, we_ref[...],
                            preferred_element_type=jnp.float32) + bemb_ref[...])


_first_call = pl.pallas_call(
    _tc_first,
    out_shape=[
        jax.ShapeDtypeStruct((NP, H), jnp.float32),
        jax.ShapeDtypeStruct((NP, 1), jnp.float32),
    ],
)

_mid_call = pl.pallas_call(
    _tc_mid,
    out_shape=jax.ShapeDtypeStruct((NP, H), jnp.float32),
)

_final_call = pl.pallas_call(
    _tc_final,
    out_shape=jax.ShapeDtypeStruct((B, EMB), jnp.float32),
)


def kernel(x, edge_index, batch, W0, b0, g0, be0, W1, b1, g1, be1,
           W2, b2, g2, be2, We, bemb):
    f32 = jnp.float32
    pad_n = NP - N
    x_p = jnp.concatenate([x, jnp.zeros((pad_n, D), f32)], axis=0)
    batch_p = jnp.concatenate(
        [batch, jnp.full((pad_n,), B, jnp.int32)]).reshape(1, NP)

    pad_e = EP - E
    pad_idx = (N + (jnp.arange(pad_e, dtype=jnp.int32) % pad_n))
    src_a = jnp.concatenate([edge_index[0], pad_idx]).reshape(NW, EC, LANE)
    dst_a = jnp.concatenate([edge_index[1], pad_idx]).reshape(NW, EC, LANE)

    z1 = jnp.zeros((NP,), f32)
    z2 = jnp.zeros((NP, H), f32)

    deg_parts = _deg_kernel(dst_a, z1)
    degp = deg_parts.reshape(NC, NP, 1)

    hs0, dinv = _first_call(x_p, W0, degp)
    agg0 = _agg_kernel(hs0, src_a, dst_a, z2)
    hs1 = _mid_call(hs0, agg0, dinv, b0.reshape(1, H), g0.reshape(1, H),
                    be0.reshape(1, H), W1)
    agg1 = _agg_kernel(hs1, src_a, dst_a, z2)
    hs2 = _mid_call(hs1, agg1, dinv, b1.reshape(1, H), g1.reshape(1, H),
                    be1.reshape(1, H), W2)
    agg2 = _agg_kernel(hs2, src_a, dst_a, z2)
    out = _final_call(hs2, agg2, dinv, b2.reshape(1, H), g2.reshape(1, H),
                      be2.reshape(1, H), batch_p, We, bemb.reshape(1, EMB))
    return out
